# Initial kernel scaffold; baseline (speedup 1.0000x reference)
import jax
import jax.numpy as jnp
from jax.experimental import pallas as pl


def kernel(x, edge_index, edge_weight, kernel):
    raise NotImplementedError("write your pallas kernel here")


# speedup vs baseline: 4.4411x; 4.4411x over previous
import jax
import jax.numpy as jnp
from jax import lax
from jax.experimental import pallas as pl
from jax.experimental.pallas import tpu as pltpu
from jax.experimental.pallas import tpu_sc as plsc

N_NODES = 10000
N_EDGES = 320000
D_FEAT = 128
UNITS = 128

NC = 2
NS = 16
LANES = 16
CHUNK = 80
EDGES_PER_TILE = N_EDGES // (NC * NS)
N_CHUNKS = EDGES_PER_TILE // CHUNK
SLAB = 640
ROWS_PAD = NS * SLAB


def _matmul_body(x_ref, w_ref, h_ref):
    h_ref[...] = jnp.dot(x_ref[...], w_ref[...],
                         preferred_element_type=jnp.float32)


def _matmul(x, w):
    bm = 1000
    return pl.pallas_call(
        _matmul_body,
        grid=(N_NODES // bm,),
        in_specs=[
            pl.BlockSpec((bm, D_FEAT), lambda j: (j, 0)),
            pl.BlockSpec((D_FEAT, UNITS), lambda j: (0, 0)),
        ],
        out_specs=pl.BlockSpec((bm, UNITS), lambda j: (j, 0)),
        out_shape=jax.ShapeDtypeStruct((N_NODES, UNITS), jnp.float32),
    )(x, w)


def _combine_body(p_ref, o_ref):
    o_ref[...] = p_ref[0] + p_ref[1]


def _combine(parts):
    bm = 1000
    return pl.pallas_call(
        _combine_body,
        grid=(N_NODES // bm,),
        in_specs=[pl.BlockSpec((NC, bm, UNITS), lambda j: (0, j, 0))],
        out_specs=pl.BlockSpec((bm, UNITS), lambda j: (j, 0)),
        out_shape=jax.ShapeDtypeStruct((N_NODES, UNITS), jnp.float32),
    )(parts)


def _edge_body(h, col, row, w, zeros, out,
               col_v, row_v, w_v, gbuf, acc_sh, sem):
    core = lax.axis_index("c")
    tid = lax.axis_index("s")

    r0 = tid * SLAB
    pltpu.sync_copy(zeros, acc_sh.at[pl.ds(r0, SLAB)])
    plsc.subcore_barrier()

    def chunk_step(c, carry):
        off = (core * NS + tid) * EDGES_PER_TILE + c * CHUNK
        pltpu.sync_copy(col.at[pl.ds(off, CHUNK)], col_v)
        pltpu.sync_copy(row.at[pl.ds(off, CHUNK)], row_v)
        pltpu.sync_copy(w.at[pl.ds(off, CHUNK)], w_v)
        pltpu.async_copy(h.at[col_v], gbuf, sem).wait()

        def group_step(g, carry2):
            wvec = w_v[pl.ds(g * LANES, LANES)]
            for l in range(LANES):
                ws = lax.gather(
                    wvec, jnp.full((LANES, 1), l, jnp.int32),
                    lax.GatherDimensionNumbers(
                        offset_dims=(), collapsed_slice_dims=(0,),
                        start_index_map=(0,)),
                    (1,), mode=lax.GatherScatterMode.PROMISE_IN_BOUNDS)
                e = g * LANES + l
                for j in range(UNITS // LANES):
                    sl = pl.ds(j * LANES, LANES)
                    gbuf[e, sl] = gbuf[e, sl] * ws
            return carry2

        lax.fori_loop(0, CHUNK // LANES, group_step, 0)
        pltpu.sync_copy(gbuf, acc_sh.at[row_v], add=True)
        return carry

    lax.fori_loop(0, N_CHUNKS, chunk_step, 0)
    plsc.subcore_barrier()

    @pl.when(tid < NS - 1)
    def _():
        pltpu.sync_copy(acc_sh.at[pl.ds(r0, SLAB)],
                        out.at[core, pl.ds(r0, SLAB)])

    @pl.when(tid == NS - 1)
    def _():
        last = (NS - 1) * SLAB
        rem = N_NODES - last
        pltpu.sync_copy(acc_sh.at[pl.ds(last, rem)],
                        out.at[core, pl.ds(last, rem)])


def _edge_kernel(h, col, row, w, zeros):
    mesh = plsc.VectorSubcoreMesh(core_axis_name="c", subcore_axis_name="s",
                                  num_cores=NC, num_subcores=NS)
    f = pl.kernel(
        _edge_body,
        out_type=jax.ShapeDtypeStruct((NC, N_NODES, UNITS), jnp.float32),
        mesh=mesh,
        scratch_types=[
            pltpu.VMEM((CHUNK,), jnp.int32),
            pltpu.VMEM((CHUNK,), jnp.int32),
            pltpu.VMEM((CHUNK,), jnp.float32),
            pltpu.VMEM((CHUNK, UNITS), jnp.float32),
            pltpu.VMEM_SHARED((ROWS_PAD, UNITS), jnp.float32),
            pltpu.SemaphoreType.DMA,
        ],
    )
    return f(h, col, row, w, zeros)


@jax.jit
def kernel(x, edge_index, edge_weight, kernel):
    ei = edge_index.astype(jnp.int32)
    row = ei[:, 0]
    col = ei[:, 1]
    h = _matmul(x, kernel)
    zeros = jnp.zeros((SLAB, UNITS), jnp.float32)
    parts = _edge_kernel(h, col, row, edge_weight, zeros)
    return _combine(parts)


# speedup vs baseline: 10.6502x; 2.3981x over previous
import jax
import jax.numpy as jnp
from jax import lax
from jax.experimental import pallas as pl
from jax.experimental.pallas import tpu as pltpu
from jax.experimental.pallas import tpu_sc as plsc

N_NODES = 10000
N_EDGES = 320000
D_FEAT = 128
UNITS = 128

NC = 2
NS = 16
LANES = 16
CHUNK = 80
EDGES_PER_TILE = N_EDGES // (NC * NS)
NCH = EDGES_PER_TILE // CHUNK
NBUF = 3
HALVES = (32, 32, 32, 29)
SLAB = 624
LAST_SLAB = N_NODES - (NS - 1) * SLAB


def _matmul_body(x_ref, w_ref, h_ref):
    h_ref[...] = jnp.dot(x_ref[...], w_ref[...],
                         preferred_element_type=jnp.float32)


def _matmul(x, w):
    bm = 1000
    return pl.pallas_call(
        _matmul_body,
        grid=(N_NODES // bm,),
        in_specs=[
            pl.BlockSpec((bm, D_FEAT), lambda j: (j, 0)),
            pl.BlockSpec((D_FEAT, UNITS), lambda j: (0, 0)),
        ],
        out_specs=pl.BlockSpec((bm, UNITS), lambda j: (j, 0)),
        out_shape=jax.ShapeDtypeStruct((N_NODES, UNITS), jnp.float32),
    )(x, w)


def _combine_body(p_ref, o_ref):
    o_ref[...] = p_ref[0] + p_ref[1]


def _combine(parts):
    bm = 1000
    return pl.pallas_call(
        _combine_body,
        grid=(N_NODES // bm,),
        in_specs=[pl.BlockSpec((NC, bm, UNITS), lambda j: (0, j, 0))],
        out_specs=pl.BlockSpec((bm, UNITS), lambda j: (j, 0)),
        out_shape=jax.ShapeDtypeStruct((N_NODES, UNITS), jnp.float32),
    )(parts)


def _edge_body(h, col3, row3, w3, zeros, out,
               col2d, row2d, w2d, g0, g1, g2, acc_sh,
               gs0, gs1, gs2, ss0, ss1, ss2):
    core = lax.axis_index("c")
    tid = lax.axis_index("s")
    gbufs = (g0, g1, g2)
    gsems = (gs0, gs1, gs2)
    ssems = (ss0, ss1, ss2)

    r0 = tid * SLAB

    @pl.when(tid < NS - 1)
    def _():
        pltpu.sync_copy(zeros.at[pl.ds(0, SLAB)], acc_sh.at[pl.ds(r0, SLAB)])

    @pl.when(tid == NS - 1)
    def _():
        pltpu.sync_copy(zeros, acc_sh.at[pl.ds((NS - 1) * SLAB, LAST_SLAB)])

    plsc.subcore_barrier()
    widx = core * NS + tid

    def gather_start(c, b):
        pltpu.async_copy(h.at[col2d.at[c]], gbufs[b], gsems[b])

    def gather_wait(c, b):
        pltpu.make_async_copy(h.at[col2d.at[c]], gbufs[b], gsems[b]).wait()

    def scatter_start(c, b):
        pltpu.async_copy(gbufs[b], acc_sh.at[row2d.at[c]], ssems[b],
                         add=True)

    def scatter_wait(c, b):
        pltpu.make_async_copy(gbufs[b], acc_sh.at[row2d.at[c]],
                              ssems[b]).wait()

    def scale(c, b):
        def group_step(g, carry):
            wvec = w2d[c, pl.ds(g * LANES, LANES)]
            for l in range(LANES):
                ws = lax.gather(
                    wvec, jnp.full((LANES, 1), l, jnp.int32),
                    lax.GatherDimensionNumbers(
                        offset_dims=(), collapsed_slice_dims=(0,),
                        start_index_map=(0,)),
                    (1,), mode=lax.GatherScatterMode.PROMISE_IN_BOUNDS)
                e = g * LANES + l
                for j in range(UNITS // LANES):
                    sl = pl.ds(j * LANES, LANES)
                    gbufs[b][e, sl] = gbufs[b][e, sl] * ws
            return carry

        lax.fori_loop(0, CHUNK // LANES, group_step, 0)

    h0 = 0
    for n_local in HALVES:
        pltpu.sync_copy(col3.at[widx, pl.ds(h0, n_local)],
                        col2d.at[pl.ds(0, n_local)])
        pltpu.sync_copy(row3.at[widx, pl.ds(h0, n_local)],
                        row2d.at[pl.ds(0, n_local)])
        pltpu.sync_copy(w3.at[widx, pl.ds(h0, n_local)],
                        w2d.at[pl.ds(0, n_local)])
        gather_start(0, 0)

        def triple(i, carry):
            for k in range(NBUF):
                c = NBUF * i + k
                b_next = (k + 1) % NBUF

                @pl.when(c >= 2)
                def _():
                    scatter_wait(c - 2, b_next)

                gather_start(c + 1, b_next)
                gather_wait(c, k)
                scale(c, k)
                scatter_start(c, k)
            return carry

        n_triples = (n_local - 2) // NBUF
        lax.fori_loop(0, n_triples, triple, 0)
        for c in range(NBUF * n_triples, n_local):
            b = c % NBUF
            if c >= 2:
                scatter_wait(c - 2, (c - 2) % NBUF)
            if c + 1 < n_local:
                gather_start(c + 1, (c + 1) % NBUF)
            gather_wait(c, b)
            scale(c, b)
            scatter_start(c, b)
        scatter_wait(n_local - 2, (n_local - 2) % NBUF)
        scatter_wait(n_local - 1, (n_local - 1) % NBUF)
        h0 += n_local

    plsc.subcore_barrier()

    @pl.when(tid < NS - 1)
    def _():
        pltpu.sync_copy(acc_sh.at[pl.ds(r0, SLAB)],
                        out.at[core, pl.ds(r0, SLAB)])

    @pl.when(tid == NS - 1)
    def _():
        last = (NS - 1) * SLAB
        pltpu.sync_copy(acc_sh.at[pl.ds(last, LAST_SLAB)],
                        out.at[core, pl.ds(last, LAST_SLAB)])


def _edge_kernel(h, col3, row3, w3, zeros):
    mesh = plsc.VectorSubcoreMesh(core_axis_name="c", subcore_axis_name="s",
                                  num_cores=NC, num_subcores=NS)
    f = pl.kernel(
        _edge_body,
        out_type=jax.ShapeDtypeStruct((NC, N_NODES, UNITS), jnp.float32),
        mesh=mesh,
        scratch_types=[
            pltpu.VMEM((max(HALVES), CHUNK), jnp.int32),
            pltpu.VMEM((max(HALVES), CHUNK), jnp.int32),
            pltpu.VMEM((max(HALVES), CHUNK), jnp.float32),
            pltpu.VMEM((CHUNK, UNITS), jnp.float32),
            pltpu.VMEM((CHUNK, UNITS), jnp.float32),
            pltpu.VMEM((CHUNK, UNITS), jnp.float32),
            pltpu.VMEM_SHARED((N_NODES, UNITS), jnp.float32),
            pltpu.SemaphoreType.DMA,
            pltpu.SemaphoreType.DMA,
            pltpu.SemaphoreType.DMA,
            pltpu.SemaphoreType.DMA,
            pltpu.SemaphoreType.DMA,
            pltpu.SemaphoreType.DMA,
        ],
    )
    return f(h, col3, row3, w3, zeros)


@jax.jit
def kernel(x, edge_index, edge_weight, kernel):
    ei = edge_index.astype(jnp.int32)
    row3 = ei[:, 0].reshape(NC * NS, NCH, CHUNK)
    col3 = ei[:, 1].reshape(NC * NS, NCH, CHUNK)
    w3 = edge_weight.reshape(NC * NS, NCH, CHUNK)
    h = _matmul(x, kernel)
    zeros = jnp.zeros((LAST_SLAB, UNITS), jnp.float32)
    parts = _edge_kernel(h, col3, row3, w3, zeros)
    return _combine(parts)


# speedup vs baseline: 11.4087x; 1.0712x over previous
import jax
import jax.numpy as jnp
from jax import lax
from jax.experimental import pallas as pl
from jax.experimental.pallas import tpu as pltpu
from jax.experimental.pallas import tpu_sc as plsc

N_NODES = 10000
N_EDGES = 320000
D_FEAT = 128
UNITS = 128

NC = 2
NS = 16
LANES = 16
CHUNK = 80
EDGES_PER_TILE = N_EDGES // (NC * NS)
NCH = EDGES_PER_TILE // CHUNK
NBUF = 3
RHALVES = (64, 61)
SLAB = 624
LAST_SLAB = N_NODES - (NS - 1) * SLAB


def _matmul_body(x_ref, w_ref, h_ref):
    h_ref[...] = jnp.dot(x_ref[...], w_ref[...],
                         preferred_element_type=jnp.float32)


def _matmul(x, w):
    bm = 1000
    return pl.pallas_call(
        _matmul_body,
        grid=(N_NODES // bm,),
        in_specs=[
            pl.BlockSpec((bm, D_FEAT), lambda j: (j, 0)),
            pl.BlockSpec((D_FEAT, UNITS), lambda j: (0, 0)),
        ],
        out_specs=pl.BlockSpec((bm, UNITS), lambda j: (j, 0)),
        out_shape=jax.ShapeDtypeStruct((N_NODES, UNITS), jnp.float32),
    )(x, w)


def _combine_body(p_ref, o_ref):
    o_ref[...] = p_ref[0] + p_ref[1]


def _combine(parts):
    bm = 1000
    return pl.pallas_call(
        _combine_body,
        grid=(N_NODES // bm,),
        in_specs=[pl.BlockSpec((NC, bm, UNITS), lambda j: (0, j, 0))],
        out_specs=pl.BlockSpec((bm, UNITS), lambda j: (j, 0)),
        out_shape=jax.ShapeDtypeStruct((N_NODES, UNITS), jnp.float32),
    )(parts)


def _edge_body(h, col1, row3, w1, zeros, out,
               row2d, c0, c1, c2, wb0, wb1, wb2, g0, g1, g2, acc_sh,
               is0, is1, is2, gs0, gs1, gs2, ss0, ss1, ss2):
    core = lax.axis_index("c")
    tid = lax.axis_index("s")
    cbufs = (c0, c1, c2)
    wbufs = (wb0, wb1, wb2)
    gbufs = (g0, g1, g2)
    isems = (is0, is1, is2)
    gsems = (gs0, gs1, gs2)
    ssems = (ss0, ss1, ss2)

    r0 = tid * SLAB

    @pl.when(tid < NS - 1)
    def _():
        pltpu.sync_copy(zeros.at[pl.ds(0, SLAB)], acc_sh.at[pl.ds(r0, SLAB)])

    @pl.when(tid == NS - 1)
    def _():
        pltpu.sync_copy(zeros, acc_sh.at[pl.ds((NS - 1) * SLAB, LAST_SLAB)])

    plsc.subcore_barrier()
    widx = core * NS + tid
    ebase = widx * EDGES_PER_TILE

    def idxw_start(ec, b):
        off = ebase + ec * CHUNK
        pltpu.async_copy(col1.at[pl.ds(off, CHUNK)], cbufs[b], isems[b])
        pltpu.async_copy(w1.at[pl.ds(off, CHUNK)], wbufs[b], isems[b])

    def idxw_wait(b):
        pltpu.make_async_copy(col1.at[pl.ds(0, CHUNK)], cbufs[b],
                              isems[b]).wait()
        pltpu.make_async_copy(w1.at[pl.ds(0, CHUNK)], wbufs[b],
                              isems[b]).wait()

    def gather_start(b):
        pltpu.async_copy(h.at[cbufs[b]], gbufs[b], gsems[b])

    def gather_wait(b):
        pltpu.make_async_copy(h.at[cbufs[b]], gbufs[b], gsems[b]).wait()

    def scatter_start(cl, b):
        pltpu.async_copy(gbufs[b], acc_sh.at[row2d.at[cl]], ssems[b],
                         add=True)

    def scatter_wait(cl, b):
        pltpu.make_async_copy(gbufs[b], acc_sh.at[row2d.at[cl]],
                              ssems[b]).wait()

    def scale(b):
        def group_step(g, carry):
            wvec = wbufs[b][pl.ds(g * LANES, LANES)]
            for l in range(LANES):
                ws = lax.gather(
                    wvec, jnp.full((LANES, 1), l, jnp.int32),
                    lax.GatherDimensionNumbers(
                        offset_dims=(), collapsed_slice_dims=(0,),
                        start_index_map=(0,)),
                    (1,), mode=lax.GatherScatterMode.PROMISE_IN_BOUNDS)
                e = g * LANES + l
                for j in range(UNITS // LANES):
                    sl = pl.ds(j * LANES, LANES)
                    gbufs[b][e, sl] = gbufs[b][e, sl] * ws
            return carry

        lax.fori_loop(0, CHUNK // LANES, group_step, 0)

    h0 = 0
    for n_local in RHALVES:
        pltpu.sync_copy(row3.at[widx, pl.ds(h0, n_local)],
                        row2d.at[pl.ds(0, n_local)])

        def body(cl, k):
            kn, kn2 = (k + 1) % NBUF, (k + 2) % NBUF
            gc = h0 + cl
            idxw_start(gc + 2, kn2)

            @pl.when(cl >= 2)
            def _():
                scatter_wait(cl - 2, kn)

            idxw_wait(kn)
            gather_start(kn)
            gather_wait(k)
            scale(k)
            scatter_start(cl, k)

        idxw_start(h0 + 0, 0)
        idxw_start(h0 + 1, 1)
        idxw_wait(0)
        gather_start(0)

        def triple(i, carry):
            for k in range(NBUF):
                body(NBUF * i + k, k)
            return carry

        n_triples = (n_local - 2) // NBUF
        lax.fori_loop(0, n_triples, triple, 0)
        for cl in range(NBUF * n_triples, n_local):
            k = cl % NBUF
            if cl + 2 < n_local:
                idxw_start(h0 + cl + 2, (cl + 2) % NBUF)
            if cl >= 2:
                scatter_wait(cl - 2, (cl - 2) % NBUF)
            if cl + 1 < n_local:
                idxw_wait((cl + 1) % NBUF)
                gather_start((cl + 1) % NBUF)
            gather_wait(k)
            scale(k)
            scatter_start(cl, k)
        scatter_wait(n_local - 2, (n_local - 2) % NBUF)
        scatter_wait(n_local - 1, (n_local - 1) % NBUF)
        h0 += n_local

    plsc.subcore_barrier()

    @pl.when(tid < NS - 1)
    def _():
        pltpu.sync_copy(acc_sh.at[pl.ds(r0, SLAB)],
                        out.at[core, pl.ds(r0, SLAB)])

    @pl.when(tid == NS - 1)
    def _():
        last = (NS - 1) * SLAB
        pltpu.sync_copy(acc_sh.at[pl.ds(last, LAST_SLAB)],
                        out.at[core, pl.ds(last, LAST_SLAB)])


def _edge_kernel(h, col1, row3, w1, zeros):
    mesh = plsc.VectorSubcoreMesh(core_axis_name="c", subcore_axis_name="s",
                                  num_cores=NC, num_subcores=NS)
    f = pl.kernel(
        _edge_body,
        out_type=jax.ShapeDtypeStruct((NC, N_NODES, UNITS), jnp.float32),
        mesh=mesh,
        scratch_types=[
            pltpu.VMEM((max(RHALVES), CHUNK), jnp.int32),
            pltpu.VMEM((CHUNK,), jnp.int32),
            pltpu.VMEM((CHUNK,), jnp.int32),
            pltpu.VMEM((CHUNK,), jnp.int32),
            pltpu.VMEM((CHUNK,), jnp.float32),
            pltpu.VMEM((CHUNK,), jnp.float32),
            pltpu.VMEM((CHUNK,), jnp.float32),
            pltpu.VMEM((CHUNK, UNITS), jnp.float32),
            pltpu.VMEM((CHUNK, UNITS), jnp.float32),
            pltpu.VMEM((CHUNK, UNITS), jnp.float32),
            pltpu.VMEM_SHARED((N_NODES, UNITS), jnp.float32),
            pltpu.SemaphoreType.DMA,
            pltpu.SemaphoreType.DMA,
            pltpu.SemaphoreType.DMA,
            pltpu.SemaphoreType.DMA,
            pltpu.SemaphoreType.DMA,
            pltpu.SemaphoreType.DMA,
            pltpu.SemaphoreType.DMA,
            pltpu.SemaphoreType.DMA,
            pltpu.SemaphoreType.DMA,
        ],
    )
    return f(h, col1, row3, w1, zeros)


@jax.jit
def kernel(x, edge_index, edge_weight, kernel):
    ei = edge_index.astype(jnp.int32)
    row3 = ei[:, 0].reshape(NC * NS, NCH, CHUNK)
    col1 = ei[:, 1]
    h = _matmul(x, kernel)
    zeros = jnp.zeros((LAST_SLAB, UNITS), jnp.float32)
    parts = _edge_kernel(h, col1, row3, edge_weight, zeros)
    return _combine(parts)


# speedup vs baseline: 11.4499x; 1.0036x over previous
import jax
import jax.numpy as jnp
from jax import lax
from jax.experimental import pallas as pl
from jax.experimental.pallas import tpu as pltpu
from jax.experimental.pallas import tpu_sc as plsc

N_NODES = 10000
N_EDGES = 320000
D_FEAT = 128
UNITS = 128

NC = 2
NS = 16
LANES = 16
CHUNK = 80
EDGES_PER_TILE = N_EDGES // (NC * NS)
NCH = EDGES_PER_TILE // CHUNK
NBUF = 3
RHALVES = (64, 61)
SLAB = 624
LAST_SLAB = N_NODES - (NS - 1) * SLAB


def _matmul_body(x_ref, w_ref, h_ref):
    h_ref[...] = jnp.dot(x_ref[...], w_ref[...],
                         preferred_element_type=jnp.float32)


def _matmul(x, w):
    bm = 1000
    return pl.pallas_call(
        _matmul_body,
        grid=(N_NODES // bm,),
        in_specs=[
            pl.BlockSpec((bm, D_FEAT), lambda j: (j, 0)),
            pl.BlockSpec((D_FEAT, UNITS), lambda j: (0, 0)),
        ],
        out_specs=pl.BlockSpec((bm, UNITS), lambda j: (j, 0)),
        out_shape=jax.ShapeDtypeStruct((N_NODES, UNITS), jnp.float32),
    )(x, w)


def _combine_body(p_ref, o_ref):
    o_ref[...] = p_ref[0] + p_ref[1]


def _combine(parts):
    bm = 1000
    return pl.pallas_call(
        _combine_body,
        grid=(N_NODES // bm,),
        in_specs=[pl.BlockSpec((NC, bm, UNITS), lambda j: (0, j, 0))],
        out_specs=pl.BlockSpec((bm, UNITS), lambda j: (j, 0)),
        out_shape=jax.ShapeDtypeStruct((N_NODES, UNITS), jnp.float32),
    )(parts)


def _edge_body(h, col1, rowa3, rowb3, w1, zeros, out,
               rowha, rowhb, c0, c1, c2, wb0, wb1, wb2, g0, g1, g2, acc_sh,
               is0, is1, is2, gs0, gs1, gs2, ss0, ss1, ss2):
    core = lax.axis_index("c")
    tid = lax.axis_index("s")
    cbufs = (c0, c1, c2)
    wbufs = (wb0, wb1, wb2)
    gbufs = (g0, g1, g2)
    isems = (is0, is1, is2)
    gsems = (gs0, gs1, gs2)
    ssems = (ss0, ss1, ss2)

    r0 = tid * SLAB

    @pl.when(tid < NS - 1)
    def _():
        pltpu.sync_copy(zeros.at[pl.ds(0, SLAB)], acc_sh.at[pl.ds(r0, SLAB)])

    @pl.when(tid == NS - 1)
    def _():
        pltpu.sync_copy(zeros, acc_sh.at[pl.ds((NS - 1) * SLAB, LAST_SLAB)])

    plsc.subcore_barrier()
    widx = core * NS + tid
    ebase = widx * EDGES_PER_TILE

    def idxw_start(ec, b):
        off = ebase + ec * CHUNK
        pltpu.async_copy(col1.at[pl.ds(off, CHUNK)], cbufs[b], isems[b])
        pltpu.async_copy(w1.at[pl.ds(off, CHUNK)], wbufs[b], isems[b])

    def idxw_wait(b):
        pltpu.make_async_copy(col1.at[pl.ds(0, CHUNK)], cbufs[b],
                              isems[b]).wait()
        pltpu.make_async_copy(w1.at[pl.ds(0, CHUNK)], wbufs[b],
                              isems[b]).wait()

    HALF_OFF = (0, 32)
    HALF_LEN = (32, 48)
    rowhs = (rowha, rowhb)

    def gather_start(b, half):
        o, n = HALF_OFF[half], HALF_LEN[half]
        pltpu.async_copy(h.at[cbufs[b].at[pl.ds(o, n)]],
                         gbufs[b].at[pl.ds(o, n)], gsems[b])

    def gather_wait(b, half):
        o, n = HALF_OFF[half], HALF_LEN[half]
        pltpu.make_async_copy(h.at[cbufs[b].at[pl.ds(o, n)]],
                              gbufs[b].at[pl.ds(o, n)], gsems[b]).wait()

    def scatter_start(cl, b, half):
        o, n = HALF_OFF[half], HALF_LEN[half]
        pltpu.async_copy(gbufs[b].at[pl.ds(o, n)],
                         acc_sh.at[rowhs[half].at[cl]], ssems[b],
                         add=True)

    def scatter_wait(cl, b):
        for half in range(2):
            o, n = HALF_OFF[half], HALF_LEN[half]
            pltpu.make_async_copy(gbufs[b].at[pl.ds(o, n)],
                                  acc_sh.at[rowhs[half].at[cl]],
                                  ssems[b]).wait()

    def scale(b, half):
        def group_step(g, carry):
            wvec = wbufs[b][pl.ds(g * LANES, LANES)]
            for l in range(LANES):
                ws = lax.gather(
                    wvec, jnp.full((LANES, 1), l, jnp.int32),
                    lax.GatherDimensionNumbers(
                        offset_dims=(), collapsed_slice_dims=(0,),
                        start_index_map=(0,)),
                    (1,), mode=lax.GatherScatterMode.PROMISE_IN_BOUNDS)
                e = g * LANES + l
                for j in range(UNITS // LANES):
                    sl = pl.ds(j * LANES, LANES)
                    gbufs[b][e, sl] = gbufs[b][e, sl] * ws
            return carry

        glo = HALF_OFF[half] // LANES
        ghi = (HALF_OFF[half] + HALF_LEN[half]) // LANES
        lax.fori_loop(glo, ghi, group_step, 0)

    h0 = 0
    for n_local in RHALVES:
        pltpu.sync_copy(rowa3.at[widx, pl.ds(h0, n_local)],
                        rowha.at[pl.ds(0, n_local)])
        pltpu.sync_copy(rowb3.at[widx, pl.ds(h0, n_local)],
                        rowhb.at[pl.ds(0, n_local)])

        def body(cl, k):
            kn, kn2 = (k + 1) % NBUF, (k + 2) % NBUF
            gc = h0 + cl
            idxw_start(gc + 2, kn2)

            @pl.when(cl >= 2)
            def _():
                scatter_wait(cl - 2, kn)

            idxw_wait(kn)
            gather_start(kn, 0)
            gather_start(kn, 1)
            gather_wait(k, 0)
            scale(k, 0)
            scatter_start(cl, k, 0)
            gather_wait(k, 1)
            scale(k, 1)
            scatter_start(cl, k, 1)

        idxw_start(h0 + 0, 0)
        idxw_start(h0 + 1, 1)
        idxw_wait(0)
        gather_start(0, 0)
        gather_start(0, 1)

        def triple(i, carry):
            for k in range(NBUF):
                body(NBUF * i + k, k)
            return carry

        n_triples = (n_local - 2) // NBUF
        lax.fori_loop(0, n_triples, triple, 0)
        for cl in range(NBUF * n_triples, n_local):
            k = cl % NBUF
            if cl + 2 < n_local:
                idxw_start(h0 + cl + 2, (cl + 2) % NBUF)
            if cl >= 2:
                scatter_wait(cl - 2, (cl - 2) % NBUF)
            if cl + 1 < n_local:
                idxw_wait((cl + 1) % NBUF)
                gather_start((cl + 1) % NBUF, 0)
                gather_start((cl + 1) % NBUF, 1)
            gather_wait(k, 0)
            scale(k, 0)
            scatter_start(cl, k, 0)
            gather_wait(k, 1)
            scale(k, 1)
            scatter_start(cl, k, 1)
        scatter_wait(n_local - 2, (n_local - 2) % NBUF)
        scatter_wait(n_local - 1, (n_local - 1) % NBUF)
        h0 += n_local

    plsc.subcore_barrier()

    @pl.when(tid < NS - 1)
    def _():
        pltpu.sync_copy(acc_sh.at[pl.ds(r0, SLAB)],
                        out.at[core, pl.ds(r0, SLAB)])

    @pl.when(tid == NS - 1)
    def _():
        last = (NS - 1) * SLAB
        pltpu.sync_copy(acc_sh.at[pl.ds(last, LAST_SLAB)],
                        out.at[core, pl.ds(last, LAST_SLAB)])


def _edge_kernel(h, col1, rowa3, rowb3, w1, zeros):
    mesh = plsc.VectorSubcoreMesh(core_axis_name="c", subcore_axis_name="s",
                                  num_cores=NC, num_subcores=NS)
    f = pl.kernel(
        _edge_body,
        out_type=jax.ShapeDtypeStruct((NC, N_NODES, UNITS), jnp.float32),
        mesh=mesh,
        scratch_types=[
            pltpu.VMEM((max(RHALVES), 32), jnp.int32),
            pltpu.VMEM((max(RHALVES), 48), jnp.int32),
            pltpu.VMEM((CHUNK,), jnp.int32),
            pltpu.VMEM((CHUNK,), jnp.int32),
            pltpu.VMEM((CHUNK,), jnp.int32),
            pltpu.VMEM((CHUNK,), jnp.float32),
            pltpu.VMEM((CHUNK,), jnp.float32),
            pltpu.VMEM((CHUNK,), jnp.float32),
            pltpu.VMEM((CHUNK, UNITS), jnp.float32),
            pltpu.VMEM((CHUNK, UNITS), jnp.float32),
            pltpu.VMEM((CHUNK, UNITS), jnp.float32),
            pltpu.VMEM_SHARED((N_NODES, UNITS), jnp.float32),
            pltpu.SemaphoreType.DMA,
            pltpu.SemaphoreType.DMA,
            pltpu.SemaphoreType.DMA,
            pltpu.SemaphoreType.DMA,
            pltpu.SemaphoreType.DMA,
            pltpu.SemaphoreType.DMA,
            pltpu.SemaphoreType.DMA,
            pltpu.SemaphoreType.DMA,
            pltpu.SemaphoreType.DMA,
        ],
    )
    return f(h, col1, rowa3, rowb3, w1, zeros)


@jax.jit
def kernel(x, edge_index, edge_weight, kernel):
    ei = edge_index.astype(jnp.int32)
    rowm = ei[:, 0].reshape(NC * NS, NCH, CHUNK)
    rowa3 = rowm[:, :, :32]
    rowb3 = rowm[:, :, 32:]
    col1 = ei[:, 1]
    h = _matmul(x, kernel)
    zeros = jnp.zeros((LAST_SLAB, UNITS), jnp.float32)
    parts = _edge_kernel(h, col1, rowa3, rowb3, edge_weight, zeros)
    return _combine(parts)


# speedup vs baseline: 11.4641x; 1.0012x over previous
import jax
import jax.numpy as jnp
from jax import lax
from jax.experimental import pallas as pl
from jax.experimental.pallas import tpu as pltpu
from jax.experimental.pallas import tpu_sc as plsc

N_NODES = 10000
N_EDGES = 320000
D_FEAT = 128
UNITS = 128

NC = 2
NS = 16
LANES = 16
CHUNK = 80
EDGES_PER_TILE = N_EDGES // (NC * NS)
NCH = EDGES_PER_TILE // CHUNK
NBUF = 3
RHALVES = (64, 61)
SLAB = 624
LAST_SLAB = N_NODES - (NS - 1) * SLAB


def _matmul_body(x_ref, w_ref, h_ref):
    h_ref[...] = jnp.dot(x_ref[...], w_ref[...],
                         preferred_element_type=jnp.float32)


def _matmul(x, w):
    bm = 1000
    return pl.pallas_call(
        _matmul_body,
        grid=(N_NODES // bm,),
        in_specs=[
            pl.BlockSpec((bm, D_FEAT), lambda j: (j, 0)),
            pl.BlockSpec((D_FEAT, UNITS), lambda j: (0, 0)),
        ],
        out_specs=pl.BlockSpec((bm, UNITS), lambda j: (j, 0)),
        out_shape=jax.ShapeDtypeStruct((N_NODES, UNITS), jnp.float32),
    )(x, w)


def _combine_body(p_ref, o_ref):
    o_ref[...] = p_ref[0] + p_ref[1]


def _combine(parts):
    bm = 1000
    return pl.pallas_call(
        _combine_body,
        grid=(N_NODES // bm,),
        in_specs=[pl.BlockSpec((NC, bm, UNITS), lambda j: (0, j, 0))],
        out_specs=pl.BlockSpec((bm, UNITS), lambda j: (j, 0)),
        out_shape=jax.ShapeDtypeStruct((N_NODES, UNITS), jnp.float32),
    )(parts)


def _edge_body(h, col1, rowa3, rowb3, w1, zeros, out,
               rowha, rowhb, c0, c1, c2, wb0, wb1, wb2, g0, g1, g2, acc_sh,
               is0, is1, is2, gs0, gs1, gs2, ss0, ss1, ss2):
    core = lax.axis_index("c")
    tid = lax.axis_index("s")
    cbufs = (c0, c1, c2)
    wbufs = (wb0, wb1, wb2)
    gbufs = (g0, g1, g2)
    isems = (is0, is1, is2)
    gsems = (gs0, gs1, gs2)
    ssems = (ss0, ss1, ss2)

    r0 = tid * SLAB

    @pl.when(tid < NS - 1)
    def _():
        pltpu.sync_copy(zeros.at[pl.ds(0, SLAB)], acc_sh.at[pl.ds(r0, SLAB)])

    @pl.when(tid == NS - 1)
    def _():
        pltpu.sync_copy(zeros, acc_sh.at[pl.ds((NS - 1) * SLAB, LAST_SLAB)])

    plsc.subcore_barrier()
    widx = core * NS + tid
    ebase = widx * EDGES_PER_TILE

    def idxw_start(ec, b):
        off = ebase + ec * CHUNK
        pltpu.async_copy(col1.at[pl.ds(off, CHUNK)], cbufs[b], isems[b])
        pltpu.async_copy(w1.at[pl.ds(off, CHUNK)], wbufs[b], isems[b])

    def idxw_wait(b):
        pltpu.make_async_copy(col1.at[pl.ds(0, CHUNK)], cbufs[b],
                              isems[b]).wait()
        pltpu.make_async_copy(w1.at[pl.ds(0, CHUNK)], wbufs[b],
                              isems[b]).wait()

    HALF_OFF = (0, 32)
    HALF_LEN = (32, 48)
    rowhs = (rowha, rowhb)

    def gather_start(b, half):
        o, n = HALF_OFF[half], HALF_LEN[half]
        pltpu.async_copy(h.at[cbufs[b].at[pl.ds(o, n)]],
                         gbufs[b].at[pl.ds(o, n)], gsems[b])

    def gather_wait(b, half):
        o, n = HALF_OFF[half], HALF_LEN[half]
        pltpu.make_async_copy(h.at[cbufs[b].at[pl.ds(o, n)]],
                              gbufs[b].at[pl.ds(o, n)], gsems[b]).wait()

    def scatter_start(cl, b, half):
        o, n = HALF_OFF[half], HALF_LEN[half]
        pltpu.async_copy(gbufs[b].at[pl.ds(o, n)],
                         acc_sh.at[rowhs[half].at[cl]], ssems[b],
                         add=True)

    def scatter_wait(cl, b):
        for half in range(2):
            o, n = HALF_OFF[half], HALF_LEN[half]
            pltpu.make_async_copy(gbufs[b].at[pl.ds(o, n)],
                                  acc_sh.at[rowhs[half].at[cl]],
                                  ssems[b]).wait()

    def scale(b, half):
        def group_step(g, carry):
            wvec = wbufs[b][pl.ds(g * LANES, LANES)]
            for l in range(LANES):
                ws = lax.gather(
                    wvec, jnp.full((LANES, 1), l, jnp.int32),
                    lax.GatherDimensionNumbers(
                        offset_dims=(), collapsed_slice_dims=(0,),
                        start_index_map=(0,)),
                    (1,), mode=lax.GatherScatterMode.PROMISE_IN_BOUNDS)
                e = g * LANES + l
                for j in range(UNITS // LANES):
                    sl = pl.ds(j * LANES, LANES)
                    gbufs[b][e, sl] = gbufs[b][e, sl] * ws
            return carry

        glo = HALF_OFF[half] // LANES
        ghi = (HALF_OFF[half] + HALF_LEN[half]) // LANES
        lax.fori_loop(glo, ghi, group_step, 0)

    h0 = 0
    for n_local in RHALVES:
        pltpu.sync_copy(rowa3.at[widx, pl.ds(h0, n_local)],
                        rowha.at[pl.ds(0, n_local)])
        pltpu.sync_copy(rowb3.at[widx, pl.ds(h0, n_local)],
                        rowhb.at[pl.ds(0, n_local)])

        def body(cl, k):
            kn, kn2 = (k + 1) % NBUF, (k + 2) % NBUF
            gc = h0 + cl
            idxw_start(gc + 2, kn2)

            @pl.when(cl >= 2)
            def _():
                scatter_wait(cl - 2, kn)

            idxw_wait(kn)
            gather_start(kn, 0)
            gather_start(kn, 1)
            gather_wait(k, 0)
            scale(k, 0)
            scatter_start(cl, k, 0)
            gather_wait(k, 1)
            scale(k, 1)
            scatter_start(cl, k, 1)

        idxw_start(h0 + 0, 0)
        idxw_start(h0 + 1, 1)
        idxw_wait(0)
        gather_start(0, 0)
        gather_start(0, 1)

        def triple(i, carry):
            for k in range(NBUF):
                body(NBUF * i + k, k)
            return carry

        n_triples = (n_local - 2) // NBUF
        lax.fori_loop(0, n_triples, triple, 0)
        for cl in range(NBUF * n_triples, n_local):
            k = cl % NBUF
            if cl + 2 < n_local:
                idxw_start(h0 + cl + 2, (cl + 2) % NBUF)
            if cl >= 2:
                scatter_wait(cl - 2, (cl - 2) % NBUF)
            if cl + 1 < n_local:
                idxw_wait((cl + 1) % NBUF)
                gather_start((cl + 1) % NBUF, 0)
                gather_start((cl + 1) % NBUF, 1)
            gather_wait(k, 0)
            scale(k, 0)
            scatter_start(cl, k, 0)
            gather_wait(k, 1)
            scale(k, 1)
            scatter_start(cl, k, 1)
        scatter_wait(n_local - 2, (n_local - 2) % NBUF)
        scatter_wait(n_local - 1, (n_local - 1) % NBUF)
        h0 += n_local

    plsc.subcore_barrier()

    @pl.when(tid < NS - 1)
    def _():
        pltpu.sync_copy(acc_sh.at[pl.ds(r0, SLAB)],
                        out.at[core, pl.ds(r0, SLAB)])

    @pl.when(tid == NS - 1)
    def _():
        last = (NS - 1) * SLAB
        pltpu.sync_copy(acc_sh.at[pl.ds(last, LAST_SLAB)],
                        out.at[core, pl.ds(last, LAST_SLAB)])


def _edge_kernel(h, col1, rowa3, rowb3, w1, zeros):
    mesh = plsc.VectorSubcoreMesh(core_axis_name="c", subcore_axis_name="s",
                                  num_cores=NC, num_subcores=NS)
    f = pl.kernel(
        _edge_body,
        out_type=jax.ShapeDtypeStruct((NC, N_NODES, UNITS), jnp.float32),
        mesh=mesh,
        scratch_types=[
            pltpu.VMEM((max(RHALVES), 32), jnp.int32),
            pltpu.VMEM((max(RHALVES), 48), jnp.int32),
            pltpu.VMEM((CHUNK,), jnp.int32),
            pltpu.VMEM((CHUNK,), jnp.int32),
            pltpu.VMEM((CHUNK,), jnp.int32),
            pltpu.VMEM((CHUNK,), jnp.float32),
            pltpu.VMEM((CHUNK,), jnp.float32),
            pltpu.VMEM((CHUNK,), jnp.float32),
            pltpu.VMEM((CHUNK, UNITS), jnp.float32),
            pltpu.VMEM((CHUNK, UNITS), jnp.float32),
            pltpu.VMEM((CHUNK, UNITS), jnp.float32),
            pltpu.VMEM_SHARED((N_NODES, UNITS), jnp.float32),
            pltpu.SemaphoreType.DMA,
            pltpu.SemaphoreType.DMA,
            pltpu.SemaphoreType.DMA,
            pltpu.SemaphoreType.DMA,
            pltpu.SemaphoreType.DMA,
            pltpu.SemaphoreType.DMA,
            pltpu.SemaphoreType.DMA,
            pltpu.SemaphoreType.DMA,
            pltpu.SemaphoreType.DMA,
        ],
    )
    return f(h, col1, rowa3, rowb3, w1, zeros)


@jax.jit
def kernel(x, edge_index, edge_weight, kernel):
    eit = edge_index.astype(jnp.int32).T
    rowm = eit[0].reshape(NC * NS, NCH, CHUNK)
    rowa3 = rowm[:, :, :32]
    rowb3 = rowm[:, :, 32:]
    col1 = eit[1]
    h = _matmul(x, kernel)
    zeros = jnp.zeros((LAST_SLAB, UNITS), jnp.float32)
    parts = _edge_kernel(h, col1, rowa3, rowb3, edge_weight, zeros)
    return _combine(parts)


# speedup vs baseline: 11.7629x; 1.0261x over previous
import jax
import jax.numpy as jnp
from jax import lax
from jax.experimental import pallas as pl
from jax.experimental.pallas import tpu as pltpu
from jax.experimental.pallas import tpu_sc as plsc

N_NODES = 10000
N_EDGES = 320000
D_FEAT = 128
UNITS = 128

NC = 2
NS = 16
LANES = 16
CHUNK = 80
EDGES_PER_TILE = N_EDGES // (NC * NS)
NCH = EDGES_PER_TILE // CHUNK
NBUF = 3
RHALVES = (64, 61)
SLAB = 624
LAST_SLAB = N_NODES - (NS - 1) * SLAB


def _matmul_body(x_ref, w_ref, h_ref):
    h_ref[...] = jnp.dot(x_ref[...], w_ref[...],
                         preferred_element_type=jnp.float32)


def _matmul(x, w):
    bm = 2000
    return pl.pallas_call(
        _matmul_body,
        grid=(N_NODES // bm,),
        in_specs=[
            pl.BlockSpec((bm, D_FEAT), lambda j: (j, 0)),
            pl.BlockSpec((D_FEAT, UNITS), lambda j: (0, 0)),
        ],
        out_specs=pl.BlockSpec((bm, UNITS), lambda j: (j, 0)),
        out_shape=jax.ShapeDtypeStruct((N_NODES, UNITS), jnp.float32),
    )(x, w)


def _combine_body(p_ref, o_ref):
    o_ref[...] = p_ref[0] + p_ref[1]


def _combine(parts):
    bm = 2000
    return pl.pallas_call(
        _combine_body,
        grid=(N_NODES // bm,),
        in_specs=[pl.BlockSpec((NC, bm, UNITS), lambda j: (0, j, 0))],
        out_specs=pl.BlockSpec((bm, UNITS), lambda j: (j, 0)),
        out_shape=jax.ShapeDtypeStruct((N_NODES, UNITS), jnp.float32),
    )(parts)


def _edge_body(h, col1, rowa3, rowb3, w1, zeros, out,
               rowha, rowhb, c0, c1, c2, wb0, wb1, wb2, g0, g1, g2, acc_sh,
               is0, is1, is2, gs0, gs1, gs2, ss0, ss1, ss2):
    core = lax.axis_index("c")
    tid = lax.axis_index("s")
    cbufs = (c0, c1, c2)
    wbufs = (wb0, wb1, wb2)
    gbufs = (g0, g1, g2)
    isems = (is0, is1, is2)
    gsems = (gs0, gs1, gs2)
    ssems = (ss0, ss1, ss2)

    r0 = tid * SLAB

    @pl.when(tid < NS - 1)
    def _():
        pltpu.sync_copy(zeros.at[pl.ds(0, SLAB)], acc_sh.at[pl.ds(r0, SLAB)])

    @pl.when(tid == NS - 1)
    def _():
        pltpu.sync_copy(zeros, acc_sh.at[pl.ds((NS - 1) * SLAB, LAST_SLAB)])

    plsc.subcore_barrier()
    widx = core * NS + tid
    ebase = widx * EDGES_PER_TILE

    def idxw_start(ec, b):
        off = ebase + ec * CHUNK
        pltpu.async_copy(col1.at[pl.ds(off, CHUNK)], cbufs[b], isems[b])
        pltpu.async_copy(w1.at[pl.ds(off, CHUNK)], wbufs[b], isems[b])

    def idxw_wait(b):
        pltpu.make_async_copy(col1.at[pl.ds(0, CHUNK)], cbufs[b],
                              isems[b]).wait()
        pltpu.make_async_copy(w1.at[pl.ds(0, CHUNK)], wbufs[b],
                              isems[b]).wait()

    HALF_OFF = (0, 32)
    HALF_LEN = (32, 48)
    rowhs = (rowha, rowhb)

    def gather_start(b, half):
        o, n = HALF_OFF[half], HALF_LEN[half]
        pltpu.async_copy(h.at[cbufs[b].at[pl.ds(o, n)]],
                         gbufs[b].at[pl.ds(o, n)], gsems[b])

    def gather_wait(b, half):
        o, n = HALF_OFF[half], HALF_LEN[half]
        pltpu.make_async_copy(h.at[cbufs[b].at[pl.ds(o, n)]],
                              gbufs[b].at[pl.ds(o, n)], gsems[b]).wait()

    def scatter_start(cl, b, half):
        o, n = HALF_OFF[half], HALF_LEN[half]
        pltpu.async_copy(gbufs[b].at[pl.ds(o, n)],
                         acc_sh.at[rowhs[half].at[cl]], ssems[b],
                         add=True)

    def scatter_wait(cl, b):
        for half in range(2):
            o, n = HALF_OFF[half], HALF_LEN[half]
            pltpu.make_async_copy(gbufs[b].at[pl.ds(o, n)],
                                  acc_sh.at[rowhs[half].at[cl]],
                                  ssems[b]).wait()

    def scale(b, half):
        def group_step(g, carry):
            wvec = wbufs[b][pl.ds(g * LANES, LANES)]
            for l in range(LANES):
                ws = lax.gather(
                    wvec, jnp.full((LANES, 1), l, jnp.int32),
                    lax.GatherDimensionNumbers(
                        offset_dims=(), collapsed_slice_dims=(0,),
                        start_index_map=(0,)),
                    (1,), mode=lax.GatherScatterMode.PROMISE_IN_BOUNDS)
                e = g * LANES + l
                for j in range(UNITS // LANES):
                    sl = pl.ds(j * LANES, LANES)
                    gbufs[b][e, sl] = gbufs[b][e, sl] * ws
            return carry

        glo = HALF_OFF[half] // LANES
        ghi = (HALF_OFF[half] + HALF_LEN[half]) // LANES
        lax.fori_loop(glo, ghi, group_step, 0)

    h0 = 0
    for n_local in RHALVES:
        pltpu.sync_copy(rowa3.at[widx, pl.ds(h0, n_local)],
                        rowha.at[pl.ds(0, n_local)])
        pltpu.sync_copy(rowb3.at[widx, pl.ds(h0, n_local)],
                        rowhb.at[pl.ds(0, n_local)])

        def body(cl, k):
            kn, kn2 = (k + 1) % NBUF, (k + 2) % NBUF
            gc = h0 + cl
            idxw_start(gc + 2, kn2)

            @pl.when(cl >= 2)
            def _():
                scatter_wait(cl - 2, kn)

            idxw_wait(kn)
            gather_start(kn, 0)
            gather_start(kn, 1)
            gather_wait(k, 0)
            scale(k, 0)
            scatter_start(cl, k, 0)
            gather_wait(k, 1)
            scale(k, 1)
            scatter_start(cl, k, 1)

        idxw_start(h0 + 0, 0)
        idxw_start(h0 + 1, 1)
        idxw_wait(0)
        gather_start(0, 0)
        gather_start(0, 1)

        def triple(i, carry):
            for k in range(NBUF):
                body(NBUF * i + k, k)
            return carry

        n_triples = (n_local - 2) // NBUF
        lax.fori_loop(0, n_triples, triple, 0)
        for cl in range(NBUF * n_triples, n_local):
            k = cl % NBUF
            if cl + 2 < n_local:
                idxw_start(h0 + cl + 2, (cl + 2) % NBUF)
            if cl >= 2:
                scatter_wait(cl - 2, (cl - 2) % NBUF)
            if cl + 1 < n_local:
                idxw_wait((cl + 1) % NBUF)
                gather_start((cl + 1) % NBUF, 0)
                gather_start((cl + 1) % NBUF, 1)
            gather_wait(k, 0)
            scale(k, 0)
            scatter_start(cl, k, 0)
            gather_wait(k, 1)
            scale(k, 1)
            scatter_start(cl, k, 1)
        scatter_wait(n_local - 2, (n_local - 2) % NBUF)
        scatter_wait(n_local - 1, (n_local - 1) % NBUF)
        h0 += n_local

    plsc.subcore_barrier()

    @pl.when(tid < NS - 1)
    def _():
        pltpu.sync_copy(acc_sh.at[pl.ds(r0, SLAB)],
                        out.at[core, pl.ds(r0, SLAB)])

    @pl.when(tid == NS - 1)
    def _():
        last = (NS - 1) * SLAB
        pltpu.sync_copy(acc_sh.at[pl.ds(last, LAST_SLAB)],
                        out.at[core, pl.ds(last, LAST_SLAB)])


def _edge_kernel(h, col1, rowa3, rowb3, w1, zeros):
    mesh = plsc.VectorSubcoreMesh(core_axis_name="c", subcore_axis_name="s",
                                  num_cores=NC, num_subcores=NS)
    f = pl.kernel(
        _edge_body,
        out_type=jax.ShapeDtypeStruct((NC, N_NODES, UNITS), jnp.float32),
        mesh=mesh,
        scratch_types=[
            pltpu.VMEM((max(RHALVES), 32), jnp.int32),
            pltpu.VMEM((max(RHALVES), 48), jnp.int32),
            pltpu.VMEM((CHUNK,), jnp.int32),
            pltpu.VMEM((CHUNK,), jnp.int32),
            pltpu.VMEM((CHUNK,), jnp.int32),
            pltpu.VMEM((CHUNK,), jnp.float32),
            pltpu.VMEM((CHUNK,), jnp.float32),
            pltpu.VMEM((CHUNK,), jnp.float32),
            pltpu.VMEM((CHUNK, UNITS), jnp.float32),
            pltpu.VMEM((CHUNK, UNITS), jnp.float32),
            pltpu.VMEM((CHUNK, UNITS), jnp.float32),
            pltpu.VMEM_SHARED((N_NODES, UNITS), jnp.float32),
            pltpu.SemaphoreType.DMA,
            pltpu.SemaphoreType.DMA,
            pltpu.SemaphoreType.DMA,
            pltpu.SemaphoreType.DMA,
            pltpu.SemaphoreType.DMA,
            pltpu.SemaphoreType.DMA,
            pltpu.SemaphoreType.DMA,
            pltpu.SemaphoreType.DMA,
            pltpu.SemaphoreType.DMA,
        ],
    )
    return f(h, col1, rowa3, rowb3, w1, zeros)


@jax.jit
def kernel(x, edge_index, edge_weight, kernel):
    ei = edge_index.astype(jnp.int32)
    rowm = ei[:, 0].reshape(NC * NS, NCH, CHUNK)
    rowa3 = rowm[:, :, :32]
    rowb3 = rowm[:, :, 32:]
    col1 = ei[:, 1]
    h = _matmul(x, kernel)
    zeros = jnp.zeros((LAST_SLAB, UNITS), jnp.float32)
    parts = _edge_kernel(h, col1, rowa3, rowb3, edge_weight, zeros)
    return _combine(parts)


# speedup vs baseline: 12.0272x; 1.0225x over previous
import jax
import jax.numpy as jnp
from jax import lax
from jax.experimental import pallas as pl
from jax.experimental.pallas import tpu as pltpu
from jax.experimental.pallas import tpu_sc as plsc

N_NODES = 10000
N_EDGES = 320000
D_FEAT = 128
UNITS = 128

NC = 2
NS = 16
LANES = 16
CHUNK = 80
EDGES_PER_TILE = N_EDGES // (NC * NS)
NCH = EDGES_PER_TILE // CHUNK
NBUF = 3
RHALVES = (64, 61)
SLAB = 624
LAST_SLAB = N_NODES - (NS - 1) * SLAB


def _matmul_body(x_ref, w_ref, h_ref):
    h_ref[...] = jnp.dot(x_ref[...], w_ref[...],
                         preferred_element_type=jnp.float32)


def _matmul(x, w):
    bm = 5000
    return pl.pallas_call(
        _matmul_body,
        grid=(N_NODES // bm,),
        in_specs=[
            pl.BlockSpec((bm, D_FEAT), lambda j: (j, 0)),
            pl.BlockSpec((D_FEAT, UNITS), lambda j: (0, 0)),
        ],
        out_specs=pl.BlockSpec((bm, UNITS), lambda j: (j, 0)),
        out_shape=jax.ShapeDtypeStruct((N_NODES, UNITS), jnp.float32),
    )(x, w)


def _combine_body(p_ref, o_ref):
    o_ref[...] = p_ref[0] + p_ref[1]


def _combine(parts):
    bm = 5000
    return pl.pallas_call(
        _combine_body,
        grid=(N_NODES // bm,),
        in_specs=[pl.BlockSpec((NC, bm, UNITS), lambda j: (0, j, 0))],
        out_specs=pl.BlockSpec((bm, UNITS), lambda j: (j, 0)),
        out_shape=jax.ShapeDtypeStruct((N_NODES, UNITS), jnp.float32),
    )(parts)


def _edge_body(h, col1, rowa3, rowb3, w1, zeros, out,
               rowha, rowhb, c0, c1, c2, wb0, wb1, wb2, g0, g1, g2, acc_sh,
               is0, is1, is2, gs0, gs1, gs2, ss0, ss1, ss2):
    core = lax.axis_index("c")
    tid = lax.axis_index("s")
    cbufs = (c0, c1, c2)
    wbufs = (wb0, wb1, wb2)
    gbufs = (g0, g1, g2)
    isems = (is0, is1, is2)
    gsems = (gs0, gs1, gs2)
    ssems = (ss0, ss1, ss2)

    r0 = tid * SLAB

    @pl.when(tid < NS - 1)
    def _():
        pltpu.sync_copy(zeros.at[pl.ds(0, SLAB)], acc_sh.at[pl.ds(r0, SLAB)])

    @pl.when(tid == NS - 1)
    def _():
        pltpu.sync_copy(zeros, acc_sh.at[pl.ds((NS - 1) * SLAB, LAST_SLAB)])

    plsc.subcore_barrier()
    widx = core * NS + tid
    ebase = widx * EDGES_PER_TILE

    def idxw_start(ec, b):
        off = ebase + ec * CHUNK
        pltpu.async_copy(col1.at[pl.ds(off, CHUNK)], cbufs[b], isems[b])
        pltpu.async_copy(w1.at[pl.ds(off, CHUNK)], wbufs[b], isems[b])

    def idxw_wait(b):
        pltpu.make_async_copy(col1.at[pl.ds(0, CHUNK)], cbufs[b],
                              isems[b]).wait()
        pltpu.make_async_copy(w1.at[pl.ds(0, CHUNK)], wbufs[b],
                              isems[b]).wait()

    HALF_OFF = (0, 32)
    HALF_LEN = (32, 48)
    rowhs = (rowha, rowhb)

    def gather_start(b, half):
        o, n = HALF_OFF[half], HALF_LEN[half]
        pltpu.async_copy(h.at[cbufs[b].at[pl.ds(o, n)]],
                         gbufs[b].at[pl.ds(o, n)], gsems[b])

    def gather_wait(b, half):
        o, n = HALF_OFF[half], HALF_LEN[half]
        pltpu.make_async_copy(h.at[cbufs[b].at[pl.ds(o, n)]],
                              gbufs[b].at[pl.ds(o, n)], gsems[b]).wait()

    def scatter_start(cl, b, half):
        o, n = HALF_OFF[half], HALF_LEN[half]
        pltpu.async_copy(gbufs[b].at[pl.ds(o, n)],
                         acc_sh.at[rowhs[half].at[cl]], ssems[b],
                         add=True)

    def scatter_wait(cl, b):
        for half in range(2):
            o, n = HALF_OFF[half], HALF_LEN[half]
            pltpu.make_async_copy(gbufs[b].at[pl.ds(o, n)],
                                  acc_sh.at[rowhs[half].at[cl]],
                                  ssems[b]).wait()

    def scale(b, half):
        def group_step(g, carry):
            wvec = wbufs[b][pl.ds(g * LANES, LANES)]
            for l in range(LANES):
                ws = lax.gather(
                    wvec, jnp.full((LANES, 1), l, jnp.int32),
                    lax.GatherDimensionNumbers(
                        offset_dims=(), collapsed_slice_dims=(0,),
                        start_index_map=(0,)),
                    (1,), mode=lax.GatherScatterMode.PROMISE_IN_BOUNDS)
                e = g * LANES + l
                for j in range(UNITS // LANES):
                    sl = pl.ds(j * LANES, LANES)
                    gbufs[b][e, sl] = gbufs[b][e, sl] * ws
            return carry

        glo = HALF_OFF[half] // LANES
        ghi = (HALF_OFF[half] + HALF_LEN[half]) // LANES
        lax.fori_loop(glo, ghi, group_step, 0)

    h0 = 0
    for n_local in RHALVES:
        pltpu.sync_copy(rowa3.at[widx, pl.ds(h0, n_local)],
                        rowha.at[pl.ds(0, n_local)])
        pltpu.sync_copy(rowb3.at[widx, pl.ds(h0, n_local)],
                        rowhb.at[pl.ds(0, n_local)])

        def body(cl, k):
            kn, kn2 = (k + 1) % NBUF, (k + 2) % NBUF
            gc = h0 + cl
            idxw_start(gc + 2, kn2)

            @pl.when(cl >= 2)
            def _():
                scatter_wait(cl - 2, kn)

            idxw_wait(kn)
            gather_start(kn, 0)
            gather_start(kn, 1)
            gather_wait(k, 0)
            scale(k, 0)
            scatter_start(cl, k, 0)
            gather_wait(k, 1)
            scale(k, 1)
            scatter_start(cl, k, 1)

        idxw_start(h0 + 0, 0)
        idxw_start(h0 + 1, 1)
        idxw_wait(0)
        gather_start(0, 0)
        gather_start(0, 1)

        def triple(i, carry):
            for k in range(NBUF):
                body(NBUF * i + k, k)
            return carry

        n_triples = (n_local - 2) // NBUF
        lax.fori_loop(0, n_triples, triple, 0)
        for cl in range(NBUF * n_triples, n_local):
            k = cl % NBUF
            if cl + 2 < n_local:
                idxw_start(h0 + cl + 2, (cl + 2) % NBUF)
            if cl >= 2:
                scatter_wait(cl - 2, (cl - 2) % NBUF)
            if cl + 1 < n_local:
                idxw_wait((cl + 1) % NBUF)
                gather_start((cl + 1) % NBUF, 0)
                gather_start((cl + 1) % NBUF, 1)
            gather_wait(k, 0)
            scale(k, 0)
            scatter_start(cl, k, 0)
            gather_wait(k, 1)
            scale(k, 1)
            scatter_start(cl, k, 1)
        scatter_wait(n_local - 2, (n_local - 2) % NBUF)
        scatter_wait(n_local - 1, (n_local - 1) % NBUF)
        h0 += n_local

    plsc.subcore_barrier()

    @pl.when(tid < NS - 1)
    def _():
        pltpu.sync_copy(acc_sh.at[pl.ds(r0, SLAB)],
                        out.at[core, pl.ds(r0, SLAB)])

    @pl.when(tid == NS - 1)
    def _():
        last = (NS - 1) * SLAB
        pltpu.sync_copy(acc_sh.at[pl.ds(last, LAST_SLAB)],
                        out.at[core, pl.ds(last, LAST_SLAB)])


def _edge_kernel(h, col1, rowa3, rowb3, w1, zeros):
    mesh = plsc.VectorSubcoreMesh(core_axis_name="c", subcore_axis_name="s",
                                  num_cores=NC, num_subcores=NS)
    f = pl.kernel(
        _edge_body,
        out_type=jax.ShapeDtypeStruct((NC, N_NODES, UNITS), jnp.float32),
        mesh=mesh,
        scratch_types=[
            pltpu.VMEM((max(RHALVES), 32), jnp.int32),
            pltpu.VMEM((max(RHALVES), 48), jnp.int32),
            pltpu.VMEM((CHUNK,), jnp.int32),
            pltpu.VMEM((CHUNK,), jnp.int32),
            pltpu.VMEM((CHUNK,), jnp.int32),
            pltpu.VMEM((CHUNK,), jnp.float32),
            pltpu.VMEM((CHUNK,), jnp.float32),
            pltpu.VMEM((CHUNK,), jnp.float32),
            pltpu.VMEM((CHUNK, UNITS), jnp.float32),
            pltpu.VMEM((CHUNK, UNITS), jnp.float32),
            pltpu.VMEM((CHUNK, UNITS), jnp.float32),
            pltpu.VMEM_SHARED((N_NODES, UNITS), jnp.float32),
            pltpu.SemaphoreType.DMA,
            pltpu.SemaphoreType.DMA,
            pltpu.SemaphoreType.DMA,
            pltpu.SemaphoreType.DMA,
            pltpu.SemaphoreType.DMA,
            pltpu.SemaphoreType.DMA,
            pltpu.SemaphoreType.DMA,
            pltpu.SemaphoreType.DMA,
            pltpu.SemaphoreType.DMA,
        ],
    )
    return f(h, col1, rowa3, rowb3, w1, zeros)


@jax.jit
def kernel(x, edge_index, edge_weight, kernel):
    ei = edge_index.astype(jnp.int32)
    rowm = ei[:, 0].reshape(NC * NS, NCH, CHUNK)
    rowa3 = rowm[:, :, :32]
    rowb3 = rowm[:, :, 32:]
    col1 = ei[:, 1]
    h = _matmul(x, kernel)
    zeros = jnp.zeros((LAST_SLAB, UNITS), jnp.float32)
    parts = _edge_kernel(h, col1, rowa3, rowb3, edge_weight, zeros)
    return _combine(parts)
